# baseline (speedup 1.0000x reference)
import functools

import jax
import jax.numpy as jnp
from jax import lax
from jax.experimental import pallas as pl
from jax.experimental.pallas import tpu as pltpu
from jax.experimental.pallas import tpu_sc as plsc


_NBLK = 2000
_EBLK = 2000

_N = 50000
_E = 800000
_NT = 32
_NPT = 1564
_NPAD = _NT * _NPT
_SLOT = 3200
_NSLOT = _E // _SLOT
_EP = _E
_SLOTB = 3216
_TRASH = _SLOT
_CNTW = 4096
_BCHUNK = 128
_DH = 64


def _bin_body(dst_hbm, eids_hbm, ldst_hbm, cnts_hbm, dstbuf, eidbuf, ldstbuf, cntbuf):
    w = lax.axis_index("s") * 2 + lax.axis_index("c")
    lo = w * _NPT
    hi = lo + _NPT
    zi = jnp.zeros((16,), jnp.int32)

    def zfill(i, _):
        eidbuf[pl.ds(i * 16, 16)] = zi
        ldstbuf[pl.ds(i * 16, 16)] = zi
        return 0

    lax.fori_loop(0, _SLOTB // 16, zfill, 0)
    iota = lax.iota(jnp.int32, 16)

    def chunk_body(c, _):
        pltpu.sync_copy(dst_hbm.at[pl.ds(pl.multiple_of(c * _SLOT, 128), _SLOT)],
                        dstbuf)

        def vec_body(j, cur):
            d = dstbuf[pl.ds(j * 16, 16)]
            u = d - lo
            mi = 1 + ((u >> 31) | ((_NPT - 1 - u) >> 31))
            rank = plsc.cumsum(mi)
            pos = (cur + rank - 1) * mi + (1 - mi) * _TRASH
            eid = c * _SLOT + j * 16 + iota
            plsc.store_scatter(eidbuf, [pos], eid)
            plsc.store_scatter(ldstbuf, [pos], u)
            return cur + rank[15]

        cur = lax.fori_loop(0, _SLOT // 16, vec_body, 0)
        of = pl.multiple_of(w * _EP + c * _SLOT, 128)
        pltpu.sync_copy(eidbuf.at[pl.ds(0, _SLOT)], eids_hbm.at[pl.ds(of, _SLOT)])
        pltpu.sync_copy(ldstbuf.at[pl.ds(0, _SLOT)], ldst_hbm.at[pl.ds(of, _SLOT)])
        cntbuf[pl.ds(c * 16, 16)] = zi + cur
        return 0

    lax.fori_loop(0, _NSLOT, chunk_body, 0)
    pltpu.sync_copy(cntbuf.at[pl.ds(0, _CNTW)],
                    cnts_hbm.at[pl.ds(pl.multiple_of(w * _CNTW, 128), _CNTW)])


_bin_edges = pl.kernel(
    _bin_body,
    out_type=[
        jax.ShapeDtypeStruct((_NT * _EP,), jnp.int32),
        jax.ShapeDtypeStruct((_NT * _EP,), jnp.int32),
        jax.ShapeDtypeStruct((_NT * _CNTW,), jnp.int32),
    ],
    mesh=plsc.VectorSubcoreMesh(core_axis_name="c", subcore_axis_name="s"),
    compiler_params=pltpu.CompilerParams(needs_layout_passes=False),
    scratch_types=[
        pltpu.VMEM((_SLOT,), jnp.int32),
        pltpu.VMEM((_SLOTB,), jnp.int32),
        pltpu.VMEM((_SLOTB,), jnp.int32),
        pltpu.VMEM((_CNTW,), jnp.int32),
    ],
)


def _scatmax_body(msg_hbm, eids_hbm, ldst_hbm, cnts_hbm, out_hbm,
                  eidv, ldstv, msgstage, accf, cntbuf, sem):
    w = lax.axis_index("s") * 2 + lax.axis_index("c")
    zf = jnp.zeros((16,), jnp.float32)

    def zfill(i, _):
        accf[pl.ds(i * 16, 16)] = zf
        return 0

    lax.fori_loop(0, _NPT * _DH // 16, zfill, 0)
    pltpu.sync_copy(cnts_hbm.at[pl.ds(pl.multiple_of(w * _CNTW, 128), _CNTW)],
                    cntbuf)

    def slot_body(s, _):
        cnt = cntbuf[pl.ds(s * 16, 16)][0]
        nch = (cnt + _BCHUNK - 1) // _BCHUNK

        def chunk_body(cb, _):
            off = cb * _BCHUNK
            ofb = pl.multiple_of(w * _EP + s * _SLOT + off, 128)
            pltpu.sync_copy(eids_hbm.at[pl.ds(ofb, _BCHUNK)], eidv)
            pltpu.sync_copy(ldst_hbm.at[pl.ds(ofb, _BCHUNK)],
                            ldstv.at[pl.ds(0, _BCHUNK)])
            pltpu.async_copy(msg_hbm.at[eidv], msgstage, sem).wait()
            trip = jnp.minimum(cnt - off, _BCHUNK)

            def edge_body(k, _):
                l = ldstv[pl.ds(k, 16)][0]
                for cc in range(_DH // 16):
                    mvec = msgstage[k, pl.ds(cc * 16, 16)]
                    avec = accf[pl.ds(l * _DH + cc * 16, 16)]
                    accf[pl.ds(l * _DH + cc * 16, 16)] = jnp.maximum(avec, mvec)
                return 0

            lax.fori_loop(0, trip, edge_body, 0)
            return 0

        lax.fori_loop(0, nch, chunk_body, 0)
        return 0

    lax.fori_loop(0, _NSLOT, slot_body, 0)
    pltpu.sync_copy(accf.at[pl.ds(0, _NPT * _DH)],
                    out_hbm.at[pl.ds(pl.multiple_of(w * _NPT * _DH, 128), _NPT * _DH)])


_scatmax = pl.kernel(
    _scatmax_body,
    out_type=jax.ShapeDtypeStruct((_NPAD * _DH,), jnp.float32),
    mesh=plsc.VectorSubcoreMesh(core_axis_name="c", subcore_axis_name="s"),
    compiler_params=pltpu.CompilerParams(needs_layout_passes=False),
    scratch_types=[
        pltpu.VMEM((_BCHUNK,), jnp.int32),
        pltpu.VMEM((_BCHUNK + 16,), jnp.int32),
        pltpu.VMEM((_BCHUNK, 2 * _DH), jnp.float32),
        pltpu.VMEM((_NPT * _DH,), jnp.float32),
        pltpu.VMEM((_CNTW,), jnp.int32),
        pltpu.SemaphoreType.DMA,
    ],
)


def _mm_body(x_ref, w_ref, b_ref, o_ref, *, act):
    y = jnp.dot(x_ref[...], w_ref[...], preferred_element_type=jnp.float32)
    y = y + b_ref[...]
    if act:
        y = jnp.maximum(y, 0.0)
    o_ref[...] = y


def _mm(x, w, b, act=True, blk=_NBLK):
    n, d = x.shape
    dout = w.shape[1]
    if n % blk != 0:
        blk = 8
        pad = (-n) % blk
        x = jnp.pad(x, ((0, pad), (0, 0)))
    np_ = x.shape[0]
    out = pl.pallas_call(
        functools.partial(_mm_body, act=act),
        grid=(np_ // blk,),
        in_specs=[
            pl.BlockSpec((blk, d), lambda i: (i, 0)),
            pl.BlockSpec((d, dout), lambda i: (0, 0)),
            pl.BlockSpec((1, dout), lambda i: (0, 0)),
        ],
        out_specs=pl.BlockSpec((blk, dout), lambda i: (i, 0)),
        out_shape=jax.ShapeDtypeStruct((np_, dout), jnp.float32),
    )(x, w, b.reshape(1, -1))
    return out[:n]


def _edge_mlp_e_body(xi_ref, xj_ref, ef_ref, wa_ref, wb_ref, wc_ref, b1_ref,
                     w2_ref, b2_ref, o_ref):
    xi = xi_ref[...]
    t = xj_ref[...] - xi
    h = jnp.dot(xi, wa_ref[...], preferred_element_type=jnp.float32)
    h = h + jnp.dot(t, wb_ref[...], preferred_element_type=jnp.float32)
    h = h + jnp.dot(ef_ref[...], wc_ref[...], preferred_element_type=jnp.float32)
    h = jnp.maximum(h + b1_ref[...], 0.0)
    y = jnp.dot(h, w2_ref[...], preferred_element_type=jnp.float32) + b2_ref[...]
    y = jnp.maximum(y, 0.0)
    o_ref[...] = jnp.concatenate([y, jnp.zeros_like(y)], axis=-1)


def _edge_mlp_body(xi_ref, xj_ref, wa_ref, wb_ref, b1_ref, w2_ref, b2_ref, o_ref):
    xi = xi_ref[...]
    t = xj_ref[...] - xi
    h = jnp.dot(xi, wa_ref[...], preferred_element_type=jnp.float32)
    h = h + jnp.dot(t, wb_ref[...], preferred_element_type=jnp.float32)
    h = jnp.maximum(h + b1_ref[...], 0.0)
    y = jnp.dot(h, w2_ref[...], preferred_element_type=jnp.float32) + b2_ref[...]
    y = jnp.maximum(y, 0.0)
    o_ref[...] = jnp.concatenate([y, jnp.zeros_like(y)], axis=-1)


def _edge_mlp(xi, xj, efeat, p):
    e, d = xi.shape
    w1 = p["l1"]["W"]
    b1 = p["l1"]["b"]
    w2 = p["l2"]["W"]
    b2 = p["l2"]["b"]
    dh = w1.shape[1]
    dout = w2.shape[1]
    wa = w1[:d]
    wb = w1[d : 2 * d]
    blk = _EBLK
    if efeat is not None:
        de = efeat.shape[1]
        wc = w1[2 * d :]
        return pl.pallas_call(
            _edge_mlp_e_body,
            grid=(e // blk,),
            in_specs=[
                pl.BlockSpec((blk, d), lambda i: (i, 0)),
                pl.BlockSpec((blk, d), lambda i: (i, 0)),
                pl.BlockSpec((blk, de), lambda i: (i, 0)),
                pl.BlockSpec((d, dh), lambda i: (0, 0)),
                pl.BlockSpec((d, dh), lambda i: (0, 0)),
                pl.BlockSpec((de, dh), lambda i: (0, 0)),
                pl.BlockSpec((1, dh), lambda i: (0, 0)),
                pl.BlockSpec((dh, dout), lambda i: (0, 0)),
                pl.BlockSpec((1, dout), lambda i: (0, 0)),
            ],
            out_specs=pl.BlockSpec((blk, 2 * dout), lambda i: (i, 0)),
            out_shape=jax.ShapeDtypeStruct((e, 2 * dout), jnp.float32),
        )(xi, xj, efeat, wa, wb, wc, b1.reshape(1, -1), w2, b2.reshape(1, -1))
    return pl.pallas_call(
        _edge_mlp_body,
        grid=(e // blk,),
        in_specs=[
            pl.BlockSpec((blk, d), lambda i: (i, 0)),
            pl.BlockSpec((blk, d), lambda i: (i, 0)),
            pl.BlockSpec((d, dh), lambda i: (0, 0)),
            pl.BlockSpec((d, dh), lambda i: (0, 0)),
            pl.BlockSpec((1, dh), lambda i: (0, 0)),
            pl.BlockSpec((dh, dout), lambda i: (0, 0)),
            pl.BlockSpec((1, dout), lambda i: (0, 0)),
        ],
        out_specs=pl.BlockSpec((blk, 2 * dout), lambda i: (i, 0)),
        out_shape=jax.ShapeDtypeStruct((e, 2 * dout), jnp.float32),
    )(xi, xj, wa, wb, b1.reshape(1, -1), w2, b2.reshape(1, -1))


def _conv(x, src, dst, p, bins, efeat=None):
    xi = jnp.take(x, dst, axis=0)
    xj = jnp.take(x, src, axis=0)
    msg = _edge_mlp(xi, xj, efeat, p)
    agg = _scatmax(msg, *bins)
    return agg.reshape(_NPAD, _DH)[: x.shape[0]]


def kernel(coordinates, adjacency, node_features, edge_features, joint_types, params):
    src = adjacency[0]
    dst = adjacency[1]
    bins = _bin_edges(adjacency[1].astype(jnp.int32))
    x = _mm(coordinates, params["hid1"]["W"], params["hid1"]["b"])
    x = _mm(x, params["hid2"]["W"], params["hid2"]["b"])
    x = jnp.concatenate([x, node_features, joint_types], axis=-1)
    x = _conv(x, src, dst, params["ece1"], bins, edge_features)
    x = _conv(x, src, dst, params["ece2"], bins, edge_features)
    x = _mm(x, params["hid3"]["W"], params["hid3"]["b"])
    x = _conv(x, src, dst, params["ec1"], bins)
    ec1_out = x
    x = _conv(x, src, dst, params["ec2"], bins)
    ec2_out = x
    x = _conv(jnp.concatenate([x, ec1_out], axis=-1), src, dst, params["ec3"], bins)
    x2 = jnp.concatenate([x, ec2_out], axis=-1)
    x1 = _mm(x2, params["hid4"]["W"], params["hid4"]["b"])
    x1 = _mm(x1, params["out1"]["W"], params["out1"]["b"])
    x2 = _mm(x2, params["hid5"]["W"], params["hid5"]["b"])
    x2 = _mm(x2, params["out2"]["W"], params["out2"]["b"])
    return (x1, x2)
